# baseline (speedup 1.0000x reference)
import functools

import jax
import jax.numpy as jnp
from jax import lax
from jax.experimental import pallas as pl
from jax.experimental.pallas import tpu as pltpu
from jax.experimental.pallas import tpu_sc as plsc

N = 10000
E = 320000
D = 128
DH = D // 2
PAIRS = N // 2

NC = 2
NS = 16
L = 16

EDGES_PER_TILE = E // NS
CHUNK = 80
NCHUNKS = EDGES_PER_TILE // CHUNK
PROWS_PER_TILE = 312
WSCALE = float(2 ** 24)

_TC_BLK = 1000


def _tc_pre_body(h_ref, wr_ref, wt_ref, b_ref, y_ref, z_ref):
    h = h_ref[...]
    y_ref[...] = jnp.dot(h, wr_ref[...], preferred_element_type=jnp.float32)
    z_ref[...] = jnp.dot(h, wt_ref[...], preferred_element_type=jnp.float32) + b_ref[...]


def _tc_pre(h, wr, wt, b):
    grid = (N // _TC_BLK,)
    return pl.pallas_call(
        _tc_pre_body,
        grid=grid,
        in_specs=[
            pl.BlockSpec((_TC_BLK, D), lambda i: (i, 0)),
            pl.BlockSpec((D, D), lambda i: (0, 0)),
            pl.BlockSpec((D, D), lambda i: (0, 0)),
            pl.BlockSpec((1, D), lambda i: (0, 0)),
        ],
        out_specs=[
            pl.BlockSpec((_TC_BLK, D), lambda i: (i, 0)),
            pl.BlockSpec((_TC_BLK, D), lambda i: (i, 0)),
        ],
        out_shape=[
            jax.ShapeDtypeStruct((N, D), jnp.float32),
            jax.ShapeDtypeStruct((N, D), jnp.float32),
        ],
    )(h, wr, wt, b.reshape(1, D))


def _tc_mid_body(p_ref, z_ref, wr_ref, wt_ref, b_ref, y_ref, z2_ref):
    h = jax.nn.relu(p_ref[...] + z_ref[...])
    y_ref[...] = jnp.dot(h, wr_ref[...], preferred_element_type=jnp.float32)
    z2_ref[...] = jnp.dot(h, wt_ref[...], preferred_element_type=jnp.float32) + b_ref[...]


def _tc_mid(p, z, wr, wt, b):
    grid = (N // _TC_BLK,)
    return pl.pallas_call(
        _tc_mid_body,
        grid=grid,
        in_specs=[
            pl.BlockSpec((_TC_BLK, D), lambda i: (i, 0)),
            pl.BlockSpec((_TC_BLK, D), lambda i: (i, 0)),
            pl.BlockSpec((D, D), lambda i: (0, 0)),
            pl.BlockSpec((D, D), lambda i: (0, 0)),
            pl.BlockSpec((1, D), lambda i: (0, 0)),
        ],
        out_specs=[
            pl.BlockSpec((_TC_BLK, D), lambda i: (i, 0)),
            pl.BlockSpec((_TC_BLK, D), lambda i: (i, 0)),
        ],
        out_shape=[
            jax.ShapeDtypeStruct((N, D), jnp.float32),
            jax.ShapeDtypeStruct((N, D), jnp.float32),
        ],
    )(p, z, wr, wt, b.reshape(1, D))


def _tc_post_body(p_ref, z_ref, o_ref):
    o_ref[...] = jax.nn.relu(p_ref[...] + z_ref[...])


def _tc_post(p, z):
    grid = (N // _TC_BLK,)
    return pl.pallas_call(
        _tc_post_body,
        grid=grid,
        in_specs=[
            pl.BlockSpec((_TC_BLK, D), lambda i: (i, 0)),
            pl.BlockSpec((_TC_BLK, D), lambda i: (i, 0)),
        ],
        out_specs=pl.BlockSpec((_TC_BLK, D), lambda i: (i, 0)),
        out_shape=jax.ShapeDtypeStruct((N, D), jnp.float32),
    )(p, z)


def _sc_agg_body(y_hbm, idx_hbm, out_hbm,
                 ibuf0, ibuf1, pb0, pb1, st, y_sh, acc_sh,
                 semg0, semg1):
    c = lax.axis_index("c")
    s = lax.axis_index("s")
    r0 = s * PROWS_PER_TILE

    def load_idx(k, ibuf):
        pltpu.sync_copy(idx_hbm.at[s, k], ibuf)

    def start_gather(ibuf, buf, sem):
        return pltpu.async_copy(y_sh.at[ibuf.at[0]], buf, sem)

    def wait_gather(buf, sem):
        pltpu.make_async_copy(y_hbm.at[0, pl.ds(0, CHUNK)], buf, sem).wait()

    def process(k, ibuf, buf):
        def grp_body(gi, _):
            w16 = ibuf[2, pl.ds(gi * L, L)].astype(jnp.float32) * (1.0 / WSCALE)
            pq16 = ibuf[3, pl.ds(gi * L, L)]
            for i in range(L):
                wb = jnp.full((L,), w16[i], dtype=jnp.float32)
                pq = pq16[i]
                offp = (pq & 1) * DH
                offq = (pq >> 1) * DH
                offnq = DH - offq
                r = gi * L + i
                for g in range(DH // L):
                    v = buf[r, pl.ds(offp + g * L, L)]
                    st[r, pl.ds(offq + g * L, L)] = v * wb
                    st[r, pl.ds(offnq + g * L, L)] = jnp.zeros((L,), jnp.float32)
            return 0
        lax.fori_loop(0, CHUNK // L, grp_body, 0)
        pltpu.sync_copy(st, acc_sh.at[ibuf.at[1]], add=True)

    def zb_body(i, _):
        for g in range(D // L):
            pb0[i, pl.ds(g * L, L)] = jnp.zeros((L,), jnp.float32)
        return 0
    lax.fori_loop(0, CHUNK, zb_body, 0)

    for k in range(3):
        pltpu.sync_copy(pb0, acc_sh.at[pl.ds(r0 + k * CHUNK, CHUNK)])

    @pl.when(s == NS - 1)
    def _():
        pltpu.sync_copy(pb0, acc_sh.at[pl.ds(r0 + 240, CHUNK)])

    @pl.when(s < NS - 1)
    def _():
        pltpu.sync_copy(pb0.at[pl.ds(0, 72)], acc_sh.at[pl.ds(r0 + 240, 72)])

    @pl.when(s == NS - 1)
    def _():
        pltpu.sync_copy(y_hbm.at[c, pl.ds(r0, 320)], y_sh.at[pl.ds(r0, 320)])

    @pl.when(s < NS - 1)
    def _():
        pltpu.sync_copy(y_hbm.at[c, pl.ds(r0, PROWS_PER_TILE)],
                        y_sh.at[pl.ds(r0, PROWS_PER_TILE)])

    plsc.subcore_barrier()

    last = NCHUNKS - 1
    load_idx(0, ibuf0)
    load_idx(1, ibuf1)
    start_gather(ibuf0, pb0, semg0)
    start_gather(ibuf1, pb1, semg1)

    def slot(k, ibuf, buf, sem):
        wait_gather(buf, sem)
        process(k, ibuf, buf)
        load_idx(jnp.minimum(k + 2, last), ibuf)
        start_gather(ibuf, buf, sem)

    def pipe_body(g, _):
        k0 = 2 * g
        slot(k0, ibuf0, pb0, semg0)
        slot(k0 + 1, ibuf1, pb1, semg1)
        return 0

    lax.fori_loop(0, NCHUNKS // 2, pipe_body, 0)

    wait_gather(pb0, semg0)
    wait_gather(pb1, semg1)

    plsc.subcore_barrier()

    ob = c * PAIRS + r0

    @pl.when(s == NS - 1)
    def _():
        pltpu.sync_copy(acc_sh.at[pl.ds(r0, 320)], out_hbm.at[pl.ds(ob, 320)])

    @pl.when(s < NS - 1)
    def _():
        pltpu.sync_copy(acc_sh.at[pl.ds(r0, PROWS_PER_TILE)],
                        out_hbm.at[pl.ds(ob, PROWS_PER_TILE)])


@functools.partial(jax.jit, static_argnames=())
def _sc_agg(y2, idx_p):
    mesh = plsc.VectorSubcoreMesh(core_axis_name="c", subcore_axis_name="s",
                                  num_cores=NC, num_subcores=NS)
    k = pl.kernel(
        _sc_agg_body,
        out_type=jax.ShapeDtypeStruct((2 * PAIRS, D), jnp.float32),
        mesh=mesh,
        scratch_types=[
            pltpu.VMEM((4, CHUNK), jnp.int32),
            pltpu.VMEM((4, CHUNK), jnp.int32),
            pltpu.VMEM((CHUNK, D), jnp.float32),
            pltpu.VMEM((CHUNK, D), jnp.float32),
            pltpu.VMEM((CHUNK, D), jnp.float32),
            pltpu.VMEM_SHARED((PAIRS, D), jnp.float32),
            pltpu.VMEM_SHARED((PAIRS, D), jnp.float32),
            pltpu.SemaphoreType.DMA,
            pltpu.SemaphoreType.DMA,
        ],
    )
    return k(y2, idx_p)


def _pack_edges(src, dst, w):
    wfix = jnp.round(w * WSCALE).astype(jnp.int32)
    pq = (src & 1) | ((dst & 1) << 1)
    rows = [src >> 1, dst >> 1, wfix, pq]
    return jnp.stack(
        [a.reshape(NS, NCHUNKS, CHUNK) for a in rows], axis=2)


def _split_y(y):
    return y.reshape(PAIRS, 2, 2, DH).transpose(2, 0, 1, 3).reshape(2, PAIRS, D)


def _unsplit_agg(out):
    o = out.reshape(2, PAIRS, 2, DH)
    return o.transpose(1, 2, 0, 3).reshape(N, D)


def kernel(x, edge_index, edge_attr, batch,
           W_rel0, b_rel0, W_root0,
           W_rel1, b_rel1, W_root1,
           W_rel2, b_rel2, W_root2):
    src = edge_index[0]
    dst = edge_index[1]
    idx_p = _pack_edges(src, dst, edge_attr)

    y0, z0 = _tc_pre(x, W_rel0, W_root0, b_rel0)
    p0 = _unsplit_agg(_sc_agg(_split_y(y0), idx_p))
    y1, z1 = _tc_mid(p0, z0, W_rel1, W_root1, b_rel1)
    p1 = _unsplit_agg(_sc_agg(_split_y(y1), idx_p))
    y2, z2 = _tc_mid(p1, z1, W_rel2, W_root2, b_rel2)
    p2 = _unsplit_agg(_sc_agg(_split_y(y2), idx_p))
    return _tc_post(p2, z2)


# speedup vs baseline: 1.8700x; 1.8700x over previous
import functools

import jax
import jax.numpy as jnp
from jax import lax
from jax.experimental import pallas as pl
from jax.experimental.pallas import tpu as pltpu
from jax.experimental.pallas import tpu_sc as plsc

N = 10000
E = 320000
D = 128

NC = 2
NS = 16
L = 16

EDGES_PER_CORE = E // NC
EDGES_PER_TILE = E // (NC * NS)
CHUNK = 80
NCHUNKS = EDGES_PER_TILE // CHUNK
ROWS_PER_TILE = 624

_TC_BLK = 1000


def _tc_pre_body(h_ref, wr_ref, wt_ref, b_ref, y_ref, z_ref):
    h = h_ref[...]
    y_ref[...] = jnp.dot(h, wr_ref[...], preferred_element_type=jnp.float32)
    z_ref[...] = jnp.dot(h, wt_ref[...], preferred_element_type=jnp.float32) + b_ref[...]


def _tc_pre(h, wr, wt, b):
    grid = (N // _TC_BLK,)
    return pl.pallas_call(
        _tc_pre_body,
        grid=grid,
        in_specs=[
            pl.BlockSpec((_TC_BLK, D), lambda i: (i, 0)),
            pl.BlockSpec((D, D), lambda i: (0, 0)),
            pl.BlockSpec((D, D), lambda i: (0, 0)),
            pl.BlockSpec((1, D), lambda i: (0, 0)),
        ],
        out_specs=[
            pl.BlockSpec((_TC_BLK, D), lambda i: (i, 0)),
            pl.BlockSpec((_TC_BLK, D), lambda i: (i, 0)),
        ],
        out_shape=[
            jax.ShapeDtypeStruct((N, D), jnp.float32),
            jax.ShapeDtypeStruct((N, D), jnp.float32),
        ],
    )(h, wr, wt, b.reshape(1, D))


def _tc_mid_body(p_ref, z_ref, wr_ref, wt_ref, b_ref, y_ref, z2_ref):
    h = jax.nn.relu(p_ref[0] + p_ref[1] + z_ref[...])
    y_ref[...] = jnp.dot(h, wr_ref[...], preferred_element_type=jnp.float32)
    z2_ref[...] = jnp.dot(h, wt_ref[...], preferred_element_type=jnp.float32) + b_ref[...]


def _tc_mid(p, z, wr, wt, b):
    grid = (N // _TC_BLK,)
    return pl.pallas_call(
        _tc_mid_body,
        grid=grid,
        in_specs=[
            pl.BlockSpec((2, _TC_BLK, D), lambda i: (0, i, 0)),
            pl.BlockSpec((_TC_BLK, D), lambda i: (i, 0)),
            pl.BlockSpec((D, D), lambda i: (0, 0)),
            pl.BlockSpec((D, D), lambda i: (0, 0)),
            pl.BlockSpec((1, D), lambda i: (0, 0)),
        ],
        out_specs=[
            pl.BlockSpec((_TC_BLK, D), lambda i: (i, 0)),
            pl.BlockSpec((_TC_BLK, D), lambda i: (i, 0)),
        ],
        out_shape=[
            jax.ShapeDtypeStruct((N, D), jnp.float32),
            jax.ShapeDtypeStruct((N, D), jnp.float32),
        ],
    )(p, z, wr, wt, b.reshape(1, D))


def _tc_post_body(p_ref, z_ref, o_ref):
    o_ref[...] = jax.nn.relu(p_ref[0] + p_ref[1] + z_ref[...])


def _tc_post(p, z):
    grid = (N // _TC_BLK,)
    return pl.pallas_call(
        _tc_post_body,
        grid=grid,
        in_specs=[
            pl.BlockSpec((2, _TC_BLK, D), lambda i: (0, i, 0)),
            pl.BlockSpec((_TC_BLK, D), lambda i: (i, 0)),
        ],
        out_specs=pl.BlockSpec((_TC_BLK, D), lambda i: (i, 0)),
        out_shape=jax.ShapeDtypeStruct((N, D), jnp.float32),
    )(p, z)


def _sc_agg_body(y_hbm, src_hbm, dst_hbm, w_hbm, out_hbm,
                 src_v, dst_v, w_v, rows_v, zbuf_v, acc_sh, sem):
    c = lax.axis_index("c")
    s = lax.axis_index("s")

    def zb_body(i, _):
        for g in range(D // L):
            zbuf_v[i, pl.ds(g * L, L)] = jnp.zeros((L,), jnp.float32)
        return 0
    lax.fori_loop(0, CHUNK, zb_body, 0)

    r0 = s * ROWS_PER_TILE
    for k in range(7):
        pltpu.sync_copy(zbuf_v, acc_sh.at[pl.ds(r0 + k * CHUNK, CHUNK)])

    @pl.when(s == NS - 1)
    def _():
        pltpu.sync_copy(zbuf_v, acc_sh.at[pl.ds(r0 + 560, CHUNK)])

    @pl.when(s < NS - 1)
    def _():
        pltpu.sync_copy(zbuf_v.at[pl.ds(0, 64)], acc_sh.at[pl.ds(r0 + 560, 64)])

    plsc.subcore_barrier()

    ebase = c * EDGES_PER_CORE + s * EDGES_PER_TILE

    def chunk_body(k, _):
        b = ebase + k * CHUNK
        pltpu.sync_copy(src_hbm.at[pl.ds(b, CHUNK)], src_v)
        pltpu.sync_copy(dst_hbm.at[pl.ds(b, CHUNK)], dst_v)
        pltpu.sync_copy(w_hbm.at[pl.ds(b, CHUNK)], w_v)
        pltpu.async_copy(y_hbm.at[src_v], rows_v, sem).wait()

        def grp_body(gi, _):
            w16 = w_v[pl.ds(gi * L, L)]
            for i in range(L):
                wb = jnp.full((L,), w16[i], dtype=jnp.float32)
                r = gi * L + i
                for g in range(D // L):
                    rows_v[r, pl.ds(g * L, L)] = rows_v[r, pl.ds(g * L, L)] * wb
            return 0
        lax.fori_loop(0, CHUNK // L, grp_body, 0)

        pltpu.sync_copy(rows_v, acc_sh.at[dst_v], add=True)
        return 0

    lax.fori_loop(0, NCHUNKS, chunk_body, 0)

    plsc.subcore_barrier()

    ob = c * N + r0

    @pl.when(s == NS - 1)
    def _():
        pltpu.sync_copy(acc_sh.at[pl.ds(r0, 640)], out_hbm.at[pl.ds(ob, 640)])

    @pl.when(s < NS - 1)
    def _():
        pltpu.sync_copy(acc_sh.at[pl.ds(r0, ROWS_PER_TILE)],
                        out_hbm.at[pl.ds(ob, ROWS_PER_TILE)])


@functools.partial(jax.jit, static_argnames=())
def _sc_agg(y, src, dst, w):
    mesh = plsc.VectorSubcoreMesh(core_axis_name="c", subcore_axis_name="s",
                                  num_cores=NC, num_subcores=NS)
    k = pl.kernel(
        _sc_agg_body,
        out_type=jax.ShapeDtypeStruct((2 * N, D), jnp.float32),
        mesh=mesh,
        scratch_types=[
            pltpu.VMEM((CHUNK,), jnp.int32),
            pltpu.VMEM((CHUNK,), jnp.int32),
            pltpu.VMEM((CHUNK,), jnp.float32),
            pltpu.VMEM((CHUNK, D), jnp.float32),
            pltpu.VMEM((CHUNK, D), jnp.float32),
            pltpu.VMEM_SHARED((N, D), jnp.float32),
            pltpu.SemaphoreType.DMA,
        ],
    )
    return k(y, src, dst, w).reshape(2, N, D)


def kernel(x, edge_index, edge_attr, batch,
           W_rel0, b_rel0, W_root0,
           W_rel1, b_rel1, W_root1,
           W_rel2, b_rel2, W_root2):
    src = edge_index[0]
    dst = edge_index[1]

    y0, z0 = _tc_pre(x, W_rel0, W_root0, b_rel0)
    p0 = _sc_agg(y0, src, dst, edge_attr)
    y1, z1 = _tc_mid(p0, z0, W_rel1, W_root1, b_rel1)
    p1 = _sc_agg(y1, src, dst, edge_attr)
    y2, z2 = _tc_mid(p1, z1, W_rel2, W_root2, b_rel2)
    p2 = _sc_agg(y2, src, dst, edge_attr)
    return _tc_post(p2, z2)


# speedup vs baseline: 2.7603x; 1.4761x over previous
import functools

import jax
import jax.numpy as jnp
from jax import lax
from jax.experimental import pallas as pl
from jax.experimental.pallas import tpu as pltpu
from jax.experimental.pallas import tpu_sc as plsc

N = 10000
E = 320000
D = 128

NC = 2
NS = 16
L = 16

EDGES_PER_CORE = E // NC
EDGES_PER_TILE = E // (NC * NS)
CHUNK = 200
NCHUNKS = EDGES_PER_TILE // CHUNK
ZROWS = 104
ROWS_PER_TILE = 624

_TC_BLK = 1000


def _tc_pre_body(h_ref, wr_ref, wt_ref, b_ref, y_ref, z_ref):
    h = h_ref[...]
    y_ref[...] = jnp.dot(h, wr_ref[...], preferred_element_type=jnp.float32)
    z_ref[...] = jnp.dot(h, wt_ref[...], preferred_element_type=jnp.float32) + b_ref[...]


def _tc_pre(h, wr, wt, b):
    grid = (N // _TC_BLK,)
    return pl.pallas_call(
        _tc_pre_body,
        grid=grid,
        in_specs=[
            pl.BlockSpec((_TC_BLK, D), lambda i: (i, 0)),
            pl.BlockSpec((D, D), lambda i: (0, 0)),
            pl.BlockSpec((D, D), lambda i: (0, 0)),
            pl.BlockSpec((1, D), lambda i: (0, 0)),
        ],
        out_specs=[
            pl.BlockSpec((_TC_BLK, D), lambda i: (i, 0)),
            pl.BlockSpec((_TC_BLK, D), lambda i: (i, 0)),
        ],
        out_shape=[
            jax.ShapeDtypeStruct((N, D), jnp.float32),
            jax.ShapeDtypeStruct((N, D), jnp.float32),
        ],
    )(h, wr, wt, b.reshape(1, D))


def _tc_mid_body(p_ref, z_ref, wr_ref, wt_ref, b_ref, y_ref, z2_ref):
    h = jax.nn.relu(p_ref[0] + p_ref[1] + z_ref[...])
    y_ref[...] = jnp.dot(h, wr_ref[...], preferred_element_type=jnp.float32)
    z2_ref[...] = jnp.dot(h, wt_ref[...], preferred_element_type=jnp.float32) + b_ref[...]


def _tc_mid(p, z, wr, wt, b):
    grid = (N // _TC_BLK,)
    return pl.pallas_call(
        _tc_mid_body,
        grid=grid,
        in_specs=[
            pl.BlockSpec((2, _TC_BLK, D), lambda i: (0, i, 0)),
            pl.BlockSpec((_TC_BLK, D), lambda i: (i, 0)),
            pl.BlockSpec((D, D), lambda i: (0, 0)),
            pl.BlockSpec((D, D), lambda i: (0, 0)),
            pl.BlockSpec((1, D), lambda i: (0, 0)),
        ],
        out_specs=[
            pl.BlockSpec((_TC_BLK, D), lambda i: (i, 0)),
            pl.BlockSpec((_TC_BLK, D), lambda i: (i, 0)),
        ],
        out_shape=[
            jax.ShapeDtypeStruct((N, D), jnp.float32),
            jax.ShapeDtypeStruct((N, D), jnp.float32),
        ],
    )(p, z, wr, wt, b.reshape(1, D))


def _tc_post_body(p_ref, z_ref, o_ref):
    o_ref[...] = jax.nn.relu(p_ref[0] + p_ref[1] + z_ref[...])


def _tc_post(p, z):
    grid = (N // _TC_BLK,)
    return pl.pallas_call(
        _tc_post_body,
        grid=grid,
        in_specs=[
            pl.BlockSpec((2, _TC_BLK, D), lambda i: (0, i, 0)),
            pl.BlockSpec((_TC_BLK, D), lambda i: (i, 0)),
        ],
        out_specs=pl.BlockSpec((_TC_BLK, D), lambda i: (i, 0)),
        out_shape=jax.ShapeDtypeStruct((N, D), jnp.float32),
    )(p, z)


def _sc_agg_body(y_hbm, src_hbm, dst_hbm, w_hbm, out_hbm,
                 src_v, dst_v, w_v, rows_v, zbuf_v, acc_sh, sem):
    c = lax.axis_index("c")
    s = lax.axis_index("s")

    def zb_body(i, _):
        for g in range(D // L):
            zbuf_v[i, pl.ds(g * L, L)] = jnp.zeros((L,), jnp.float32)
        return 0
    lax.fori_loop(0, ZROWS, zb_body, 0)

    r0 = s * ROWS_PER_TILE
    for k in range(6):
        pltpu.sync_copy(zbuf_v, acc_sh.at[pl.ds(r0 + k * ZROWS, ZROWS)])

    @pl.when(s == NS - 1)
    def _():
        pltpu.sync_copy(zbuf_v.at[pl.ds(0, 16)], acc_sh.at[pl.ds(r0 + 624, 16)])

    plsc.subcore_barrier()

    ebase = c * EDGES_PER_CORE + s * EDGES_PER_TILE

    def chunk_body(k, _):
        b = ebase + k * CHUNK
        pltpu.sync_copy(src_hbm.at[pl.ds(b, CHUNK)], src_v)
        pltpu.sync_copy(dst_hbm.at[pl.ds(b, CHUNK)], dst_v)
        pltpu.sync_copy(w_hbm.at[pl.ds(b, CHUNK)], w_v.at[pl.ds(0, CHUNK)])
        pltpu.async_copy(y_hbm.at[src_v], rows_v, sem).wait()

        def grp_body(gi, _):
            w16 = w_v[pl.ds(gi * L, L)]
            for i in range(L):
                wb = jnp.full((L,), w16[i], dtype=jnp.float32)
                r = gi * L + i
                for g in range(D // L):
                    rows_v[r, pl.ds(g * L, L)] = rows_v[r, pl.ds(g * L, L)] * wb
            return 0
        lax.fori_loop(0, CHUNK // L, grp_body, 0)
        w16t = w_v[pl.ds((CHUNK // L) * L, L)]
        for i in range(CHUNK - (CHUNK // L) * L):
            wb = jnp.full((L,), w16t[i], dtype=jnp.float32)
            r = (CHUNK // L) * L + i
            for g in range(D // L):
                rows_v[r, pl.ds(g * L, L)] = rows_v[r, pl.ds(g * L, L)] * wb

        pltpu.sync_copy(rows_v, acc_sh.at[dst_v], add=True)
        return 0

    lax.fori_loop(0, NCHUNKS, chunk_body, 0)

    plsc.subcore_barrier()

    ob = c * N + r0

    @pl.when(s == NS - 1)
    def _():
        pltpu.sync_copy(acc_sh.at[pl.ds(r0, 640)], out_hbm.at[pl.ds(ob, 640)])

    @pl.when(s < NS - 1)
    def _():
        pltpu.sync_copy(acc_sh.at[pl.ds(r0, ROWS_PER_TILE)],
                        out_hbm.at[pl.ds(ob, ROWS_PER_TILE)])


@functools.partial(jax.jit, static_argnames=())
def _sc_agg(y, src, dst, w):
    mesh = plsc.VectorSubcoreMesh(core_axis_name="c", subcore_axis_name="s",
                                  num_cores=NC, num_subcores=NS)
    k = pl.kernel(
        _sc_agg_body,
        out_type=jax.ShapeDtypeStruct((2 * N, D), jnp.float32),
        mesh=mesh,
        scratch_types=[
            pltpu.VMEM((CHUNK,), jnp.int32),
            pltpu.VMEM((CHUNK,), jnp.int32),
            pltpu.VMEM((CHUNK + 8,), jnp.float32),
            pltpu.VMEM((CHUNK, D), jnp.float32),
            pltpu.VMEM((ZROWS, D), jnp.float32),
            pltpu.VMEM_SHARED((N, D), jnp.float32),
            pltpu.SemaphoreType.DMA,
        ],
    )
    return k(y, src, dst, w).reshape(2, N, D)


def kernel(x, edge_index, edge_attr, batch,
           W_rel0, b_rel0, W_root0,
           W_rel1, b_rel1, W_root1,
           W_rel2, b_rel2, W_root2):
    src = edge_index[0]
    dst = edge_index[1]

    y0, z0 = _tc_pre(x, W_rel0, W_root0, b_rel0)
    p0 = _sc_agg(y0, src, dst, edge_attr)
    y1, z1 = _tc_mid(p0, z0, W_rel1, W_root1, b_rel1)
    p1 = _sc_agg(y1, src, dst, edge_attr)
    y2, z2 = _tc_mid(p1, z1, W_rel2, W_root2, b_rel2)
    p2 = _sc_agg(y2, src, dst, edge_attr)
    return _tc_post(p2, z2)
